# baseline (speedup 1.0000x reference)
import functools

import jax
import jax.numpy as jnp
from jax import lax
from jax.experimental import pallas as pl
from jax.experimental.pallas import tpu as pltpu
from jax.experimental.pallas import tpu_sc as plsc

_K = 20
_LANES = 16
_G = 128


_DN0 = (((0,), (0,)), ((), ()))


def _topk_kernel(n, c, xs_ref, x_ref, wt_ref, idx_ref, pt_ref, qt_ref):
    xrow = xs_ref[0]
    xb = x_ref[0]
    w1 = wt_ref[:c, :]
    w2 = wt_ref[c:, :]
    qt_ref[...] = lax.dot_general(xrow, w1, _DN0,
                                  preferred_element_type=jnp.float32)
    pt_ref[...] = lax.dot_general(xrow, w1 + w2, _DN0,
                                  preferred_element_type=jnp.float32)
    s = 2.0 * lax.dot_general(xrow, xb, _DN0,
                              preferred_element_type=jnp.float32)
    s = s - jnp.sum(xb * xb, axis=0, keepdims=True)
    y = lax.bitcast_convert_type(s, jnp.int32)
    colc = lax.broadcasted_iota(jnp.int32, s.shape, 1)
    key = lax.bitcast_convert_type(
        (y & jnp.int32(~(n - 1))) | (jnp.int32(n - 1) - colc), jnp.float32)
    base = pl.program_id(0) * n
    neginf = jnp.float32(-jnp.inf)
    nm1 = jnp.int32(n - 1)
    for k in range(_K):
        m = jnp.max(key, axis=1, keepdims=True)
        mb = lax.bitcast_convert_type(m, jnp.int32)
        idx_ref[0, :, k:k + 1] = (nm1 - (mb & nm1)) + base
        key = jnp.where(key == m, neginf, key)


def _make_sc_gather(bn, out_c, nc, ns):
    nw = nc * ns
    pw = bn // nw
    ch = 32
    nchunk = pw // ch
    rpc = ch * _K
    ng = rpc // _G
    nv = out_c // _LANES
    mesh = plsc.VectorSubcoreMesh(
        core_axis_name="c", subcore_axis_name="s", num_cores=nc)

    @functools.partial(
        pl.kernel, mesh=mesh,
        compiler_params=pltpu.CompilerParams(use_tc_tiling_on_sc=False),
        out_type=(
            jax.ShapeDtypeStruct((bn, out_c), jnp.float32),
            jax.ShapeDtypeStruct((bn, out_c), jnp.float32),
            jax.ShapeDtypeStruct((nw, 2 * out_c), jnp.float32),
        ),
        scratch_types=[
            pltpu.VMEM((2 * rpc,), jnp.int32),
            pltpu.VMEM((2 * rpc, out_c), jnp.float32),
            pltpu.VMEM((2 * ch, out_c), jnp.float32),
            pltpu.VMEM((2 * ch, out_c), jnp.float32),
            pltpu.VMEM((2 * ch, out_c), jnp.float32),
            pltpu.VMEM((2 * out_c,), jnp.float32),
            pltpu.SemaphoreType.DMA,
            pltpu.SemaphoreType.DMA,
            pltpu.SemaphoreType.DMA,
            pltpu.SemaphoreType.DMA,
            pltpu.SemaphoreType.DMA,
        ],
    )
    def sc_gather(idx_hbm, qt_hbm, pt_hbm, minq_hbm, maxq_hbm, part_hbm,
                  idx_v, rows_v, p_v, min_v, max_v, acc_v,
                  sem_i, sem_p, sem_g, sem_o0, sem_o1):
        wid = lax.axis_index("s") * nc + lax.axis_index("c")
        for i in range(2 * nv):
            acc_v[pl.ds(i * _LANES, _LANES)] = jnp.zeros((_LANES,), jnp.float32)

        def fire_idx(cb):
            par = (cb % 2) * rpc
            row0 = wid * pw + cb * ch
            pltpu.async_copy(idx_hbm.at[pl.ds(row0 * _K, rpc)],
                             idx_v.at[pl.ds(par, rpc)], sem_i)

        def wait_idx(cb):
            par = (cb % 2) * rpc
            row0 = wid * pw + cb * ch
            pltpu.make_async_copy(idx_hbm.at[pl.ds(row0 * _K, rpc)],
                                  idx_v.at[pl.ds(par, rpc)], sem_i).wait()

        def fire_p(cb):
            par = (cb % 2) * ch
            row0 = wid * pw + cb * ch
            pltpu.async_copy(pt_hbm.at[pl.ds(row0, ch)],
                             p_v.at[pl.ds(par, ch)], sem_p)

        def wait_p(cb):
            par = (cb % 2) * ch
            row0 = wid * pw + cb * ch
            pltpu.make_async_copy(pt_hbm.at[pl.ds(row0, ch)],
                                  p_v.at[pl.ds(par, ch)], sem_p).wait()

        def fire_gathers(cb):
            par = (cb % 2) * rpc
            for j in range(ng):
                pltpu.async_copy(
                    qt_hbm.at[idx_v.at[pl.ds(par + j * _G, _G)]],
                    rows_v.at[pl.ds(par + j * _G, _G)], sem_g)

        def wait_gathers(cb):
            par = (cb % 2) * rpc
            pltpu.make_async_copy(qt_hbm.at[pl.ds(0, rpc)],
                                  rows_v.at[pl.ds(par, rpc)], sem_g).wait()

        def fire_out(cb):
            par = (cb % 2) * ch
            row0 = wid * pw + cb * ch

            @pl.when(cb % 2 == 0)
            def _():
                pltpu.async_copy(min_v.at[pl.ds(par, ch)],
                                 minq_hbm.at[pl.ds(row0, ch)], sem_o0)
                pltpu.async_copy(max_v.at[pl.ds(par, ch)],
                                 maxq_hbm.at[pl.ds(row0, ch)], sem_o0)

            @pl.when(cb % 2 == 1)
            def _():
                pltpu.async_copy(min_v.at[pl.ds(par, ch)],
                                 minq_hbm.at[pl.ds(row0, ch)], sem_o1)
                pltpu.async_copy(max_v.at[pl.ds(par, ch)],
                                 maxq_hbm.at[pl.ds(row0, ch)], sem_o1)

        def wait_out(cb):
            par = (cb % 2) * ch
            row0 = wid * pw + cb * ch

            @pl.when(cb % 2 == 0)
            def _():
                pltpu.make_async_copy(min_v.at[pl.ds(par, ch)],
                                      minq_hbm.at[pl.ds(row0, ch)],
                                      sem_o0).wait()
                pltpu.make_async_copy(max_v.at[pl.ds(par, ch)],
                                      maxq_hbm.at[pl.ds(row0, ch)],
                                      sem_o0).wait()

            @pl.when(cb % 2 == 1)
            def _():
                pltpu.make_async_copy(min_v.at[pl.ds(par, ch)],
                                      minq_hbm.at[pl.ds(row0, ch)],
                                      sem_o1).wait()
                pltpu.make_async_copy(max_v.at[pl.ds(par, ch)],
                                      maxq_hbm.at[pl.ds(row0, ch)],
                                      sem_o1).wait()

        fire_idx(0)
        wait_idx(0)
        fire_gathers(0)
        fire_idx(1)
        fire_p(0)

        def chunk_body(cb, carry):
            par = cb % 2
            row0 = wid * pw + cb * ch
            wait_gathers(cb)
            wait_p(cb)

            @pl.when(cb + 1 < nchunk)
            def _():
                wait_idx(cb + 1)
                fire_gathers(cb + 1)
                fire_p(cb + 1)

            @pl.when(cb + 2 < nchunk)
            def _():
                fire_idx(cb + 2)

            @pl.when(cb >= 2)
            def _():
                wait_out(cb - 2)

            def point_body(p, carry2):
                rb = par * rpc + p * _K
                pb = par * ch + p
                for ci in range(nv):
                    sl = pl.ds(ci * _LANES, _LANES)
                    v = rows_v[rb, sl]
                    mn = v
                    mx = v
                    sv = v
                    qv = v * v
                    for kk in range(1, _K):
                        v = rows_v[rb + kk, sl]
                        mn = jnp.minimum(mn, v)
                        mx = jnp.maximum(mx, v)
                        sv = sv + v
                        qv = qv + v * v
                    min_v[pb, sl] = mn
                    max_v[pb, sl] = mx
                    pv = p_v[pb, sl]
                    a0 = pl.ds(ci * _LANES, _LANES)
                    a1 = pl.ds(out_c + ci * _LANES, _LANES)
                    kf = jnp.float32(_K)
                    acc_v[a0] = acc_v[a0] + (kf * pv - sv)
                    acc_v[a1] = acc_v[a1] + (kf * pv * pv - 2.0 * pv * sv + qv)
                return carry2

            lax.fori_loop(0, ch, point_body, 0)
            fire_out(cb)
            return carry

        lax.fori_loop(0, nchunk, chunk_body, 0)
        wait_out(nchunk - 2)
        wait_out(nchunk - 1)
        pltpu.sync_copy(acc_v, part_hbm.at[wid])

    return sc_gather


def _final_kernel(count, out_c, pt_ref, mn_ref, mx_ref, part_ref,
                  g_ref, b_ref, out_ref):
    part = part_ref[...]
    s0 = jnp.sum(part[:, :out_c], axis=0, keepdims=True)
    s1 = jnp.sum(part[:, out_c:], axis=0, keepdims=True)
    mean = s0 / count
    var = s1 / count - mean * mean
    rstd = lax.rsqrt(var + 1e-5)
    scale = g_ref[...] * rstd
    shift = b_ref[...] - mean * scale
    p = pt_ref[...]
    z = jnp.where(scale >= 0.0, p - mn_ref[...], p - mx_ref[...])
    o = z * scale + shift
    o = jnp.where(o > 0.0, o, 0.2 * o)
    out_ref[0] = jnp.transpose(o, (1, 0))


def kernel(x, W, gamma, beta):
    b, c, n = x.shape
    out_c = W.shape[0]
    bn = b * n

    info = plsc.get_sparse_core_info()
    nc, ns = info.num_cores, info.num_subcores
    nw = nc * ns

    rt = 512
    nt = n // rt
    nh = 4
    bh = b // nh
    wt = W.T
    halves = []
    parts = []
    for h in range(nh):
        b0 = h * bh
        idxg, pt, qt = pl.pallas_call(
            functools.partial(_topk_kernel, n, c),
            grid=(bh, nt),
            in_specs=[
                pl.BlockSpec((1, c, rt), lambda bi, t, b0=b0: (b0 + bi, 0, t)),
                pl.BlockSpec((1, c, n), lambda bi, t, b0=b0: (b0 + bi, 0, 0)),
                pl.BlockSpec((2 * c, out_c), lambda bi, t: (0, 0)),
            ],
            out_specs=[
                pl.BlockSpec((1, rt, _K), lambda bi, t: (bi, t, 0)),
                pl.BlockSpec((rt, out_c), lambda bi, t: (bi * nt + t, 0)),
                pl.BlockSpec((rt, out_c), lambda bi, t: (bi * nt + t, 0)),
            ],
            out_shape=[
                jax.ShapeDtypeStruct((bh, n, _K), jnp.int32),
                jax.ShapeDtypeStruct((bh * n, out_c), jnp.float32),
                jax.ShapeDtypeStruct((bh * n, out_c), jnp.float32),
            ],
        )(x, x, wt)
        idx_r = idxg.reshape(bh * n * _K)
        minq, maxq, part = _make_sc_gather(bh * n, out_c, nc, ns)(
            idx_r, qt, pt)
        halves.append((pt, minq, maxq))
        parts.append(part)

    part_all = jnp.concatenate(parts, axis=0)
    outs = []
    for h in range(nh):
        pt, minq, maxq = halves[h]
        outs.append(pl.pallas_call(
            functools.partial(_final_kernel, float(bn * _K), out_c),
            grid=(bh,),
            in_specs=[
                pl.BlockSpec((n, out_c), lambda i: (i, 0)),
                pl.BlockSpec((n, out_c), lambda i: (i, 0)),
                pl.BlockSpec((n, out_c), lambda i: (i, 0)),
                pl.BlockSpec((nh * nw, 2 * out_c), lambda i: (0, 0)),
                pl.BlockSpec((1, out_c), lambda i: (0, 0)),
                pl.BlockSpec((1, out_c), lambda i: (0, 0)),
            ],
            out_specs=pl.BlockSpec((1, out_c, n), lambda i: (i, 0, 0)),
            out_shape=jax.ShapeDtypeStruct((bh, out_c, n), jnp.float32),
        )(pt, minq, maxq, part_all,
          gamma.reshape(1, out_c), beta.reshape(1, out_c)))
    return jnp.concatenate(outs, axis=0)


# speedup vs baseline: 1.0175x; 1.0175x over previous
import functools

import jax
import jax.numpy as jnp
from jax import lax
from jax.experimental import pallas as pl
from jax.experimental.pallas import tpu as pltpu
from jax.experimental.pallas import tpu_sc as plsc

_K = 20
_LANES = 16
_G = 128


_DN0 = (((0,), (0,)), ((), ()))


def _topk_kernel(n, c, xs_ref, x_ref, wt_ref, idx_ref, pt_ref, qt_ref):
    xrow = xs_ref[0]
    xb = x_ref[0]
    w1 = wt_ref[:c, :]
    w2 = wt_ref[c:, :]
    qt_ref[...] = lax.dot_general(xrow, w1, _DN0,
                                  preferred_element_type=jnp.float32)
    pt_ref[...] = lax.dot_general(xrow, w1 + w2, _DN0,
                                  preferred_element_type=jnp.float32)
    s = 2.0 * lax.dot_general(xrow, xb, _DN0,
                              preferred_element_type=jnp.float32)
    s = s - jnp.sum(xb * xb, axis=0, keepdims=True)
    y = lax.bitcast_convert_type(s, jnp.int32)
    colc = lax.broadcasted_iota(jnp.int32, s.shape, 1)
    key = lax.bitcast_convert_type(
        (y & jnp.int32(~(n - 1))) | (jnp.int32(n - 1) - colc), jnp.float32)
    base = pl.program_id(0) * n
    neginf = jnp.float32(-jnp.inf)
    nm1 = jnp.int32(n - 1)
    for k in range(_K):
        m = jnp.max(key, axis=1, keepdims=True)
        mb = lax.bitcast_convert_type(m, jnp.int32)
        idx_ref[0, :, k:k + 1] = (nm1 - (mb & nm1)) + base
        key = jnp.where(key == m, neginf, key)


def _make_sc_gather(bn, out_c, nc, ns):
    nw = nc * ns
    pw = bn // nw
    ch = 32
    nchunk = pw // ch
    rpc = ch * _K
    ng = rpc // _G
    nv = out_c // _LANES
    mesh = plsc.VectorSubcoreMesh(
        core_axis_name="c", subcore_axis_name="s", num_cores=nc)

    @functools.partial(
        pl.kernel, mesh=mesh,
        compiler_params=pltpu.CompilerParams(use_tc_tiling_on_sc=False),
        out_type=(
            jax.ShapeDtypeStruct((bn, out_c), jnp.float32),
            jax.ShapeDtypeStruct((bn, out_c), jnp.float32),
            jax.ShapeDtypeStruct((nw, 2 * out_c), jnp.float32),
        ),
        scratch_types=[
            pltpu.VMEM((2 * rpc,), jnp.int32),
            pltpu.VMEM((2 * rpc, out_c), jnp.float32),
            pltpu.VMEM((2 * ch, out_c), jnp.float32),
            pltpu.VMEM((2 * ch, out_c), jnp.float32),
            pltpu.VMEM((2 * ch, out_c), jnp.float32),
            pltpu.VMEM((2 * out_c,), jnp.float32),
            pltpu.SemaphoreType.DMA,
            pltpu.SemaphoreType.DMA,
            pltpu.SemaphoreType.DMA,
            pltpu.SemaphoreType.DMA,
            pltpu.SemaphoreType.DMA,
        ],
    )
    def sc_gather(idx_hbm, qt_hbm, pt_hbm, minq_hbm, maxq_hbm, part_hbm,
                  idx_v, rows_v, p_v, min_v, max_v, acc_v,
                  sem_i, sem_p, sem_g, sem_o0, sem_o1):
        wid = lax.axis_index("s") * nc + lax.axis_index("c")
        for i in range(2 * nv):
            acc_v[pl.ds(i * _LANES, _LANES)] = jnp.zeros((_LANES,), jnp.float32)

        def fire_idx(cb):
            par = (cb % 2) * rpc
            row0 = wid * pw + cb * ch
            pltpu.async_copy(idx_hbm.at[pl.ds(row0 * _K, rpc)],
                             idx_v.at[pl.ds(par, rpc)], sem_i)

        def wait_idx(cb):
            par = (cb % 2) * rpc
            row0 = wid * pw + cb * ch
            pltpu.make_async_copy(idx_hbm.at[pl.ds(row0 * _K, rpc)],
                                  idx_v.at[pl.ds(par, rpc)], sem_i).wait()

        def fire_p(cb):
            par = (cb % 2) * ch
            row0 = wid * pw + cb * ch
            pltpu.async_copy(pt_hbm.at[pl.ds(row0, ch)],
                             p_v.at[pl.ds(par, ch)], sem_p)

        def wait_p(cb):
            par = (cb % 2) * ch
            row0 = wid * pw + cb * ch
            pltpu.make_async_copy(pt_hbm.at[pl.ds(row0, ch)],
                                  p_v.at[pl.ds(par, ch)], sem_p).wait()

        def fire_gathers(cb):
            par = (cb % 2) * rpc
            for j in range(ng):
                pltpu.async_copy(
                    qt_hbm.at[idx_v.at[pl.ds(par + j * _G, _G)]],
                    rows_v.at[pl.ds(par + j * _G, _G)], sem_g)

        def wait_gathers(cb):
            par = (cb % 2) * rpc
            pltpu.make_async_copy(qt_hbm.at[pl.ds(0, rpc)],
                                  rows_v.at[pl.ds(par, rpc)], sem_g).wait()

        def fire_out(cb):
            par = (cb % 2) * ch
            row0 = wid * pw + cb * ch

            @pl.when(cb % 2 == 0)
            def _():
                pltpu.async_copy(min_v.at[pl.ds(par, ch)],
                                 minq_hbm.at[pl.ds(row0, ch)], sem_o0)
                pltpu.async_copy(max_v.at[pl.ds(par, ch)],
                                 maxq_hbm.at[pl.ds(row0, ch)], sem_o0)

            @pl.when(cb % 2 == 1)
            def _():
                pltpu.async_copy(min_v.at[pl.ds(par, ch)],
                                 minq_hbm.at[pl.ds(row0, ch)], sem_o1)
                pltpu.async_copy(max_v.at[pl.ds(par, ch)],
                                 maxq_hbm.at[pl.ds(row0, ch)], sem_o1)

        def wait_out(cb):
            par = (cb % 2) * ch
            row0 = wid * pw + cb * ch

            @pl.when(cb % 2 == 0)
            def _():
                pltpu.make_async_copy(min_v.at[pl.ds(par, ch)],
                                      minq_hbm.at[pl.ds(row0, ch)],
                                      sem_o0).wait()
                pltpu.make_async_copy(max_v.at[pl.ds(par, ch)],
                                      maxq_hbm.at[pl.ds(row0, ch)],
                                      sem_o0).wait()

            @pl.when(cb % 2 == 1)
            def _():
                pltpu.make_async_copy(min_v.at[pl.ds(par, ch)],
                                      minq_hbm.at[pl.ds(row0, ch)],
                                      sem_o1).wait()
                pltpu.make_async_copy(max_v.at[pl.ds(par, ch)],
                                      maxq_hbm.at[pl.ds(row0, ch)],
                                      sem_o1).wait()

        fire_idx(0)
        wait_idx(0)
        fire_gathers(0)
        fire_idx(1)
        fire_p(0)

        def chunk_body(cb, carry):
            par = cb % 2
            row0 = wid * pw + cb * ch
            wait_gathers(cb)
            wait_p(cb)

            @pl.when(cb + 1 < nchunk)
            def _():
                wait_idx(cb + 1)
                fire_gathers(cb + 1)
                fire_p(cb + 1)

            @pl.when(cb + 2 < nchunk)
            def _():
                fire_idx(cb + 2)

            @pl.when(cb >= 2)
            def _():
                wait_out(cb - 2)

            def point_body(p, carry2):
                rb = par * rpc + p * _K
                pb = par * ch + p
                for ci in range(nv):
                    sl = pl.ds(ci * _LANES, _LANES)
                    v = rows_v[rb, sl]
                    mn = v
                    mx = v
                    sv = v
                    qv = v * v
                    for kk in range(1, _K):
                        v = rows_v[rb + kk, sl]
                        mn = jnp.minimum(mn, v)
                        mx = jnp.maximum(mx, v)
                        sv = sv + v
                        qv = qv + v * v
                    min_v[pb, sl] = mn
                    max_v[pb, sl] = mx
                    pv = p_v[pb, sl]
                    a0 = pl.ds(ci * _LANES, _LANES)
                    a1 = pl.ds(out_c + ci * _LANES, _LANES)
                    kf = jnp.float32(_K)
                    acc_v[a0] = acc_v[a0] + (kf * pv - sv)
                    acc_v[a1] = acc_v[a1] + (kf * pv * pv - 2.0 * pv * sv + qv)
                return carry2

            lax.fori_loop(0, ch, point_body, 0)
            fire_out(cb)
            return carry

        lax.fori_loop(0, nchunk, chunk_body, 0)
        wait_out(nchunk - 2)
        wait_out(nchunk - 1)
        pltpu.sync_copy(acc_v, part_hbm.at[wid])

    return sc_gather


def _final_kernel(count, out_c, pt_ref, mn_ref, mx_ref, part_ref,
                  g_ref, b_ref, out_ref):
    part = part_ref[...]
    s0 = jnp.sum(part[:, :out_c], axis=0, keepdims=True)
    s1 = jnp.sum(part[:, out_c:], axis=0, keepdims=True)
    mean = s0 / count
    var = s1 / count - mean * mean
    rstd = lax.rsqrt(var + 1e-5)
    scale = g_ref[...] * rstd
    shift = b_ref[...] - mean * scale
    p = pt_ref[...]
    z = jnp.where(scale >= 0.0, p - mn_ref[...], p - mx_ref[...])
    o = z * scale + shift
    o = jnp.where(o > 0.0, o, 0.2 * o)
    out_ref[0] = jnp.transpose(o, (1, 0))


def kernel(x, W, gamma, beta):
    b, c, n = x.shape
    out_c = W.shape[0]
    bn = b * n

    info = plsc.get_sparse_core_info()
    nc, ns = info.num_cores, info.num_subcores
    nw = nc * ns

    rt = 512
    nt = n // rt
    nh = 2
    bh = b // nh
    wt = W.T
    halves = []
    parts = []
    for h in range(nh):
        b0 = h * bh
        idxg, pt, qt = pl.pallas_call(
            functools.partial(_topk_kernel, n, c),
            grid=(bh, nt),
            in_specs=[
                pl.BlockSpec((1, c, rt), lambda bi, t, b0=b0: (b0 + bi, 0, t)),
                pl.BlockSpec((1, c, n), lambda bi, t, b0=b0: (b0 + bi, 0, 0)),
                pl.BlockSpec((2 * c, out_c), lambda bi, t: (0, 0)),
            ],
            out_specs=[
                pl.BlockSpec((1, rt, _K), lambda bi, t: (bi, t, 0)),
                pl.BlockSpec((rt, out_c), lambda bi, t: (bi * nt + t, 0)),
                pl.BlockSpec((rt, out_c), lambda bi, t: (bi * nt + t, 0)),
            ],
            out_shape=[
                jax.ShapeDtypeStruct((bh, n, _K), jnp.int32),
                jax.ShapeDtypeStruct((bh * n, out_c), jnp.float32),
                jax.ShapeDtypeStruct((bh * n, out_c), jnp.float32),
            ],
        )(x, x, wt)
        idx_r = idxg.reshape(bh * n * _K)
        minq, maxq, part = _make_sc_gather(bh * n, out_c, nc, ns)(
            idx_r, qt, pt)
        halves.append((pt, minq, maxq))
        parts.append(part)

    part_all = jnp.concatenate(parts, axis=0)
    outs = []
    for h in range(nh):
        pt, minq, maxq = halves[h]
        outs.append(pl.pallas_call(
            functools.partial(_final_kernel, float(bn * _K), out_c),
            grid=(bh,),
            in_specs=[
                pl.BlockSpec((n, out_c), lambda i: (i, 0)),
                pl.BlockSpec((n, out_c), lambda i: (i, 0)),
                pl.BlockSpec((n, out_c), lambda i: (i, 0)),
                pl.BlockSpec((nh * nw, 2 * out_c), lambda i: (0, 0)),
                pl.BlockSpec((1, out_c), lambda i: (0, 0)),
                pl.BlockSpec((1, out_c), lambda i: (0, 0)),
            ],
            out_specs=pl.BlockSpec((1, out_c, n), lambda i: (i, 0, 0)),
            out_shape=jax.ShapeDtypeStruct((bh, out_c, n), jnp.float32),
        )(pt, minq, maxq, part_all,
          gamma.reshape(1, out_c), beta.reshape(1, out_c)))
    return jnp.concatenate(outs, axis=0)
